# Initial kernel scaffold; baseline (speedup 1.0000x reference)
import jax
import jax.numpy as jnp
from jax.experimental import pallas as pl


def kernel(time, xt, W_proj, b_proj, freqs, gene_table, mol_table, dose_table, assay_table, cell_table, exp_table, well_table, assay_idx, cell_type_idx, experiment_idx, well_idx, gene_pert_idx, mol_pert_idx, dose_idx):
    raise NotImplementedError("write your pallas kernel here")


# speedup vs baseline: 1.4136x; 1.4136x over previous
import functools

import jax
import jax.numpy as jnp
from jax import lax
from jax.experimental import pallas as pl
from jax.experimental.pallas import tpu as pltpu
from jax.experimental.pallas import tpu_sc as plsc

B = 4096
DATA_DIM = 512
PROJ_DIM = 1024
T_DIM = 128
PERT_DIM = 256
COV_DIM = 64

NC = 2
NS = 16
NW = NC * NS

PB = (3 * B) // NW
CB = B // NW
PCH = PB // 128


def _sc_body(gene_t, mol_t, dose_t, assay_t, cell_t, exp_t, well_t,
             gi, mi, di, ai, ci, ei, wi,
             go, mo, do_, ao, co, eo, wo,
             idx3, idx1, r256, r64, sem):
    wid = lax.axis_index("s") * NC + lax.axis_index("c")

    for (tbl, ihbm, ohbm) in ((gene_t, gi, go), (mol_t, mi, mo)):
        pltpu.sync_copy(ihbm.at[pl.ds(wid * PB, PB)], idx3)
        cps = [
            pltpu.async_copy(tbl.at[idx3.at[pl.ds(j * 128, 128)]],
                             r256.at[pl.ds(j * 128, 128)], sem)
            for j in range(PCH)
        ]
        for c in cps:
            c.wait()
        pltpu.sync_copy(r256, ohbm.at[pl.ds(wid * PB, PB)])

    pltpu.sync_copy(di.at[pl.ds(wid * PB, PB)], idx3)
    cps = [
        pltpu.async_copy(dose_t.at[idx3.at[pl.ds(j * 128, 128)]],
                         r64.at[pl.ds(j * 128, 128)], sem)
        for j in range(PCH)
    ]
    for c in cps:
        c.wait()
    pltpu.sync_copy(r64, do_.at[pl.ds(wid * PB, PB)])

    for (tbl, ihbm, ohbm) in ((assay_t, ai, ao), (cell_t, ci, co),
                              (exp_t, ei, eo), (well_t, wi, wo)):
        pltpu.sync_copy(ihbm.at[pl.ds(wid * CB, CB)], idx1)
        pltpu.async_copy(tbl.at[idx1], r64.at[pl.ds(0, 128)], sem).wait()
        pltpu.sync_copy(r64.at[pl.ds(0, 128)],
                        ohbm.at[pl.ds(wid * CB, CB)])


_sc_gather = pl.kernel(
    _sc_body,
    out_type=(
        jax.ShapeDtypeStruct((3 * B, PERT_DIM), jnp.float32),
        jax.ShapeDtypeStruct((3 * B, PERT_DIM), jnp.float32),
        jax.ShapeDtypeStruct((3 * B, COV_DIM), jnp.float32),
        jax.ShapeDtypeStruct((B, COV_DIM), jnp.float32),
        jax.ShapeDtypeStruct((B, COV_DIM), jnp.float32),
        jax.ShapeDtypeStruct((B, COV_DIM), jnp.float32),
        jax.ShapeDtypeStruct((B, COV_DIM), jnp.float32),
    ),
    mesh=plsc.VectorSubcoreMesh(core_axis_name="c", subcore_axis_name="s"),
    scratch_types=[
        pltpu.VMEM((PB,), jnp.int32),
        pltpu.VMEM((CB,), jnp.int32),
        pltpu.VMEM((PB, PERT_DIM), jnp.float32),
        pltpu.VMEM((PB, COV_DIM), jnp.float32),
        pltpu.SemaphoreType.DMA,
    ],
    compiler_params=pltpu.CompilerParams(use_tc_tiling_on_sc=False),
)


def _tc_body(time_ref, freqs_ref, xt_ref, w_ref, b_ref,
             time_out, xt_out):
    t = time_ref[...]
    f = freqs_ref[...]
    time_out[...] = jnp.sin((2.0 * jnp.pi) * t * f)
    xt_out[...] = jnp.dot(
        xt_ref[...], w_ref[...],
        preferred_element_type=jnp.float32,
        precision=lax.Precision.HIGHEST,
    ) + b_ref[...]


BT = 512

_tc_dense = pl.pallas_call(
    _tc_body,
    grid=(B // BT,),
    in_specs=[
        pl.BlockSpec((BT, 1), lambda i: (i, 0)),
        pl.BlockSpec((1, T_DIM), lambda i: (0, 0)),
        pl.BlockSpec((BT, DATA_DIM), lambda i: (i, 0)),
        pl.BlockSpec((DATA_DIM, PROJ_DIM), lambda i: (0, 0)),
        pl.BlockSpec((1, PROJ_DIM), lambda i: (0, 0)),
    ],
    out_specs=[
        pl.BlockSpec((BT, T_DIM), lambda i: (i, 0)),
        pl.BlockSpec((BT, PROJ_DIM), lambda i: (i, 0)),
    ],
    out_shape=[
        jax.ShapeDtypeStruct((B, T_DIM), jnp.float32),
        jax.ShapeDtypeStruct((B, PROJ_DIM), jnp.float32),
    ],
)


def kernel(time, xt, W_proj, b_proj, freqs, gene_table, mol_table,
           dose_table, assay_table, cell_table, exp_table, well_table,
           assay_idx, cell_type_idx, experiment_idx, well_idx,
           gene_pert_idx, mol_pert_idx, dose_idx):
    gene_o, mol_o, dose_o, assay_o, cell_o, exp_o, well_o = _sc_gather(
        gene_table, mol_table, dose_table, assay_table, cell_table,
        exp_table, well_table, gene_pert_idx, mol_pert_idx, dose_idx,
        assay_idx, cell_type_idx, experiment_idx, well_idx)

    time_emb, xt_emb = _tc_dense(
        time.reshape(B, 1), freqs.reshape(1, T_DIM), xt, W_proj,
        b_proj.reshape(1, PROJ_DIM))

    return (time_emb, xt_emb, assay_o, cell_o, exp_o, well_o,
            gene_o.reshape(3, B, PERT_DIM),
            mol_o.reshape(3, B, PERT_DIM),
            dose_o.reshape(3, B, COV_DIM))


# speedup vs baseline: 1.5728x; 1.1127x over previous
import jax
import jax.numpy as jnp
from jax import lax
from jax.experimental import pallas as pl
from jax.experimental.pallas import tpu as pltpu
from jax.experimental.pallas import tpu_sc as plsc

B = 4096
DATA_DIM = 512
PROJ_DIM = 1024
T_DIM = 128
PERT_DIM = 256
COV_DIM = 64

NC = 2
NS = 16
NW = NC * NS

PB = (3 * B) // NW
CHUNK = 128
NCH = (2 * PB) // CHUNK
RING = 3


def _sc_body(gene_t, mol_t, gi, mi, go, mo, idx, rbuf, sem_g, sem_o):
    wid = lax.axis_index("s") * NC + lax.axis_index("c")

    pltpu.sync_copy(gi.at[pl.ds(wid * PB, PB)], idx.at[pl.ds(0, PB)])
    pltpu.sync_copy(mi.at[pl.ds(wid * PB, PB)], idx.at[pl.ds(PB, PB)])

    def gather(k):
        tbl = gene_t if k < NCH // 2 else mol_t
        return pltpu.async_copy(
            tbl.at[idx.at[pl.ds(k * CHUNK, CHUNK)]],
            rbuf.at[k % RING], sem_g)

    def writeback(k):
        ohbm = go if k < NCH // 2 else mo
        base = (wid * PB) + (k % (NCH // 2)) * CHUNK
        return pltpu.async_copy(
            rbuf.at[k % RING], ohbm.at[pl.ds(base, CHUNK)], sem_o)

    gcp = [None] * NCH
    ocp = [None] * NCH
    for k in range(RING):
        gcp[k] = gather(k)
    for k in range(NCH):
        gcp[k].wait()
        ocp[k] = writeback(k)
        if k + RING < NCH:
            ocp[k].wait()
            gcp[k + RING] = gather(k + RING)
    for k in range(NCH - RING, NCH):
        ocp[k].wait()


_sc_gather = pl.kernel(
    _sc_body,
    out_type=(
        jax.ShapeDtypeStruct((3 * B, PERT_DIM), jnp.float32),
        jax.ShapeDtypeStruct((3 * B, PERT_DIM), jnp.float32),
    ),
    mesh=plsc.VectorSubcoreMesh(core_axis_name="c", subcore_axis_name="s"),
    scratch_types=[
        pltpu.VMEM((2 * PB,), jnp.int32),
        pltpu.VMEM((RING, CHUNK, PERT_DIM), jnp.float32),
        pltpu.SemaphoreType.DMA,
        pltpu.SemaphoreType.DMA,
    ],
)


BT = 512


def _onehot_rows(idx_col, table):
    v = table.shape[0]
    iota = lax.broadcasted_iota(jnp.int32, (idx_col.shape[0], v), 1)
    oh = (iota == idx_col).astype(jnp.float32)
    return jnp.dot(oh, table, preferred_element_type=jnp.float32,
                   precision=lax.Precision.HIGHEST)


def _tc_body(time_ref, freqs_ref, xt_ref, w_ref, b_ref,
             assay_t, cell_t, exp_t, well_t, dose_t,
             ai_ref, ci_ref, ei_ref, wi_ref, di_ref,
             time_out, xt_out, assay_out, cell_out, exp_out, well_out,
             dose_out):
    t = time_ref[...]
    f = freqs_ref[...]
    time_out[...] = jnp.sin((2.0 * jnp.pi) * t * f)
    xt_out[...] = jnp.dot(
        xt_ref[...], w_ref[...],
        preferred_element_type=jnp.float32,
        precision=lax.Precision.HIGHEST,
    ) + b_ref[...]

    assay_out[...] = _onehot_rows(ai_ref[...], assay_t[...])
    cell_out[...] = _onehot_rows(ci_ref[...], cell_t[...])
    exp_out[...] = _onehot_rows(ei_ref[...], exp_t[...])
    well_out[...] = _onehot_rows(wi_ref[...], well_t[...])

    d = di_ref[...]
    dose_out[...] = jnp.stack(
        [_onehot_rows(d[s], dose_t[...]) for s in range(3)], axis=0)


_tc_dense = pl.pallas_call(
    _tc_body,
    grid=(B // BT,),
    in_specs=[
        pl.BlockSpec((BT, 1), lambda i: (i, 0)),
        pl.BlockSpec((1, T_DIM), lambda i: (0, 0)),
        pl.BlockSpec((BT, DATA_DIM), lambda i: (i, 0)),
        pl.BlockSpec((DATA_DIM, PROJ_DIM), lambda i: (0, 0)),
        pl.BlockSpec((1, PROJ_DIM), lambda i: (0, 0)),
        pl.BlockSpec((128, COV_DIM), lambda i: (0, 0)),
        pl.BlockSpec((64, COV_DIM), lambda i: (0, 0)),
        pl.BlockSpec((256, COV_DIM), lambda i: (0, 0)),
        pl.BlockSpec((1536, COV_DIM), lambda i: (0, 0)),
        pl.BlockSpec((256, COV_DIM), lambda i: (0, 0)),
        pl.BlockSpec((BT, 1), lambda i: (i, 0)),
        pl.BlockSpec((BT, 1), lambda i: (i, 0)),
        pl.BlockSpec((BT, 1), lambda i: (i, 0)),
        pl.BlockSpec((BT, 1), lambda i: (i, 0)),
        pl.BlockSpec((3, BT, 1), lambda i: (0, i, 0)),
    ],
    out_specs=[
        pl.BlockSpec((BT, T_DIM), lambda i: (i, 0)),
        pl.BlockSpec((BT, PROJ_DIM), lambda i: (i, 0)),
        pl.BlockSpec((BT, COV_DIM), lambda i: (i, 0)),
        pl.BlockSpec((BT, COV_DIM), lambda i: (i, 0)),
        pl.BlockSpec((BT, COV_DIM), lambda i: (i, 0)),
        pl.BlockSpec((BT, COV_DIM), lambda i: (i, 0)),
        pl.BlockSpec((3, BT, COV_DIM), lambda i: (0, i, 0)),
    ],
    out_shape=[
        jax.ShapeDtypeStruct((B, T_DIM), jnp.float32),
        jax.ShapeDtypeStruct((B, PROJ_DIM), jnp.float32),
        jax.ShapeDtypeStruct((B, COV_DIM), jnp.float32),
        jax.ShapeDtypeStruct((B, COV_DIM), jnp.float32),
        jax.ShapeDtypeStruct((B, COV_DIM), jnp.float32),
        jax.ShapeDtypeStruct((B, COV_DIM), jnp.float32),
        jax.ShapeDtypeStruct((3, B, COV_DIM), jnp.float32),
    ],
)


def kernel(time, xt, W_proj, b_proj, freqs, gene_table, mol_table,
           dose_table, assay_table, cell_table, exp_table, well_table,
           assay_idx, cell_type_idx, experiment_idx, well_idx,
           gene_pert_idx, mol_pert_idx, dose_idx):
    gene_o, mol_o = _sc_gather(gene_table, mol_table,
                               gene_pert_idx, mol_pert_idx)

    (time_emb, xt_emb, assay_o, cell_o, exp_o, well_o, dose_o) = _tc_dense(
        time.reshape(B, 1), freqs.reshape(1, T_DIM), xt, W_proj,
        b_proj.reshape(1, PROJ_DIM),
        assay_table, cell_table, exp_table, well_table, dose_table,
        assay_idx.reshape(B, 1), cell_type_idx.reshape(B, 1),
        experiment_idx.reshape(B, 1), well_idx.reshape(B, 1),
        dose_idx.reshape(3, B, 1))

    return (time_emb, xt_emb, assay_o, cell_o, exp_o, well_o,
            gene_o.reshape(3, B, PERT_DIM),
            mol_o.reshape(3, B, PERT_DIM),
            dose_o)


# speedup vs baseline: 2.4202x; 1.5387x over previous
import jax
import jax.numpy as jnp
from jax import lax
from jax.experimental import pallas as pl
from jax.experimental.pallas import tpu as pltpu
from jax.experimental.pallas import tpu_sc as plsc

B = 4096
DATA_DIM = 512
PROJ_DIM = 1024
T_DIM = 128
PERT_DIM = 256
COV_DIM = 64

NC = 2
NS = 16
NW = NC * NS

PB = (3 * B) // NW
CB = B // NW
CHUNK = 128
NCH = (2 * PB) // CHUNK
RING = 2
NNCH = PB // CHUNK + 4
NRING = 2
PAD_DIM = 128

IDX_LEN = 3 * PB + 4 * CB


def _sc_body(gene_t, mol_t, dose_t, assay_t, cell_t, exp_t, well_t,
             gi, mi, di, ai, ci, ei, wi,
             go, mo, do_, ao, co, eo, wo,
             idx, rbuf, nbuf, sem_g, sem_o, sem_ng, sem_no):
    wid = lax.axis_index("s") * NC + lax.axis_index("c")

    pltpu.sync_copy(gi.at[pl.ds(wid * PB, PB)], idx.at[pl.ds(0, PB)])
    pltpu.sync_copy(mi.at[pl.ds(wid * PB, PB)], idx.at[pl.ds(PB, PB)])
    pltpu.sync_copy(di.at[pl.ds(wid * PB, PB)], idx.at[pl.ds(2 * PB, PB)])
    pltpu.sync_copy(ai.at[pl.ds(wid * CB, CB)], idx.at[pl.ds(3 * PB, CB)])
    pltpu.sync_copy(ci.at[pl.ds(wid * CB, CB)],
                    idx.at[pl.ds(3 * PB + CB, CB)])
    pltpu.sync_copy(ei.at[pl.ds(wid * CB, CB)],
                    idx.at[pl.ds(3 * PB + 2 * CB, CB)])
    pltpu.sync_copy(wi.at[pl.ds(wid * CB, CB)],
                    idx.at[pl.ds(3 * PB + 3 * CB, CB)])

    def gather(k):
        tbl = gene_t if k < NCH // 2 else mol_t
        return pltpu.async_copy(
            tbl.at[idx.at[pl.ds(k * CHUNK, CHUNK)]],
            rbuf.at[k % RING], sem_g)

    def writeback(k):
        ohbm = go if k < NCH // 2 else mo
        base = (wid * PB) + (k % (NCH // 2)) * CHUNK
        return pltpu.async_copy(
            rbuf.at[k % RING], ohbm.at[pl.ds(base, CHUNK)], sem_o)

    narrow = (
        [(dose_t, 2 * PB + k * CHUNK, do_, wid * PB + k * CHUNK)
         for k in range(PB // CHUNK)]
        + [(assay_t, 3 * PB, ao, wid * CB),
           (cell_t, 3 * PB + CB, co, wid * CB),
           (exp_t, 3 * PB + 2 * CB, eo, wid * CB),
           (well_t, 3 * PB + 3 * CB, wo, wid * CB)]
    )

    def ngather(k):
        tbl, ioff, _, _ = narrow[k]
        return pltpu.async_copy(
            tbl.at[idx.at[pl.ds(ioff, CHUNK)]],
            nbuf.at[k % NRING], sem_ng)

    def nwriteback(k):
        _, _, ohbm, obase = narrow[k]
        return pltpu.async_copy(
            nbuf.at[k % NRING], ohbm.at[pl.ds(obase, CHUNK)], sem_no)

    gcp = [None] * NCH
    ocp = [None] * NCH
    ngc = [None] * NNCH
    noc = [None] * NNCH

    for k in range(NRING):
        ngc[k] = ngather(k)
    for k in range(RING):
        gcp[k] = gather(k)

    for k in range(NCH):
        gcp[k].wait()
        ocp[k] = writeback(k)
        if k + RING < NCH:
            ocp[k].wait()
            gcp[k + RING] = gather(k + RING)

    for k in range(NNCH):
        ngc[k].wait()
        noc[k] = nwriteback(k)
        if k + NRING < NNCH:
            noc[k].wait()
            ngc[k + NRING] = ngather(k + NRING)

    for k in range(NCH - RING, NCH):
        ocp[k].wait()
    for k in range(NNCH - NRING, NNCH):
        noc[k].wait()


_sc_gather = pl.kernel(
    _sc_body,
    out_type=(
        jax.ShapeDtypeStruct((3 * B, PERT_DIM), jnp.float32),
        jax.ShapeDtypeStruct((3 * B, PERT_DIM), jnp.float32),
        jax.ShapeDtypeStruct((3 * B, PAD_DIM), jnp.float32),
        jax.ShapeDtypeStruct((B, PAD_DIM), jnp.float32),
        jax.ShapeDtypeStruct((B, PAD_DIM), jnp.float32),
        jax.ShapeDtypeStruct((B, PAD_DIM), jnp.float32),
        jax.ShapeDtypeStruct((B, PAD_DIM), jnp.float32),
    ),
    mesh=plsc.VectorSubcoreMesh(core_axis_name="c", subcore_axis_name="s"),
    scratch_types=[
        pltpu.VMEM((IDX_LEN,), jnp.int32),
        pltpu.VMEM((RING, CHUNK, PERT_DIM), jnp.float32),
        pltpu.VMEM((NRING, CHUNK, PAD_DIM), jnp.float32),
        pltpu.SemaphoreType.DMA,
        pltpu.SemaphoreType.DMA,
        pltpu.SemaphoreType.DMA,
        pltpu.SemaphoreType.DMA,
    ],
)


BT = 512


def _tc_body(time_ref, freqs_ref, xt_ref, w_ref, b_ref, time_out, xt_out):
    t = time_ref[...]
    f = freqs_ref[...]
    time_out[...] = jnp.sin((2.0 * jnp.pi) * t * f)
    xt_out[...] = jnp.dot(
        xt_ref[...], w_ref[...],
        preferred_element_type=jnp.float32,
    ) + b_ref[...]


_tc_dense = pl.pallas_call(
    _tc_body,
    grid=(B // BT,),
    in_specs=[
        pl.BlockSpec((BT, 1), lambda i: (i, 0)),
        pl.BlockSpec((1, T_DIM), lambda i: (0, 0)),
        pl.BlockSpec((BT, DATA_DIM), lambda i: (i, 0)),
        pl.BlockSpec((DATA_DIM, PROJ_DIM), lambda i: (0, 0)),
        pl.BlockSpec((1, PROJ_DIM), lambda i: (0, 0)),
    ],
    out_specs=[
        pl.BlockSpec((BT, T_DIM), lambda i: (i, 0)),
        pl.BlockSpec((BT, PROJ_DIM), lambda i: (i, 0)),
    ],
    out_shape=[
        jax.ShapeDtypeStruct((B, T_DIM), jnp.float32),
        jax.ShapeDtypeStruct((B, PROJ_DIM), jnp.float32),
    ],
)


def kernel(time, xt, W_proj, b_proj, freqs, gene_table, mol_table,
           dose_table, assay_table, cell_table, exp_table, well_table,
           assay_idx, cell_type_idx, experiment_idx, well_idx,
           gene_pert_idx, mol_pert_idx, dose_idx):
    pad = [(0, 0), (0, PAD_DIM - COV_DIM)]
    (gene_o, mol_o, dose_o, assay_o, cell_o, exp_o, well_o) = _sc_gather(
        gene_table, mol_table,
        jnp.pad(dose_table, pad), jnp.pad(assay_table, pad),
        jnp.pad(cell_table, pad), jnp.pad(exp_table, pad),
        jnp.pad(well_table, pad),
        gene_pert_idx, mol_pert_idx, dose_idx,
        assay_idx, cell_type_idx, experiment_idx, well_idx)

    time_emb, xt_emb = _tc_dense(
        time.reshape(B, 1), freqs.reshape(1, T_DIM), xt, W_proj,
        b_proj.reshape(1, PROJ_DIM))

    return (time_emb, xt_emb,
            assay_o[:, :COV_DIM], cell_o[:, :COV_DIM],
            exp_o[:, :COV_DIM], well_o[:, :COV_DIM],
            gene_o.reshape(3, B, PERT_DIM),
            mol_o.reshape(3, B, PERT_DIM),
            dose_o[:, :COV_DIM].reshape(3, B, COV_DIM))
